# Initial kernel scaffold; baseline (speedup 1.0000x reference)
import jax
import jax.numpy as jnp
from jax.experimental import pallas as pl


def kernel(embeddings, weights, neighbor_set, WQ, bQ, WK, bK, WV, bV, W1, b1, gamma, beta, moving_mean, moving_var):
    raise NotImplementedError("write your pallas kernel here")


# speedup vs baseline: 2.0382x; 2.0382x over previous
import functools

import jax
import jax.numpy as jnp
from jax import lax
from jax.experimental import pallas as pl
from jax.experimental.pallas import tpu as pltpu
from jax.experimental.pallas import tpu_sc as plsc

N = 4096
K = 32
D = 128
H = 128
NW = 32
NPW = N // NW
GROUP = 16
NGROUPS = NPW // GROUP
ROWS = GROUP * K
CHUNK = 128
NCHUNK = NPW * K // CHUNK


def _leaky(x):
    return jnp.where(x >= 0, x, 0.3 * x)


def _project_body(e_ref, wq_ref, bqc_ref, wk_ref, bk_ref, wv_ref, bv_ref,
                  eqt_ref, ek_ref, ev_ref):
    e = e_ref[...]
    ek_ref[...] = _leaky(
        jnp.dot(e, wk_ref[...], preferred_element_type=jnp.float32) + bk_ref[...])
    ev_ref[...] = _leaky(
        jnp.dot(e, wv_ref[...], preferred_element_type=jnp.float32) + bv_ref[...])
    eqt = lax.dot_general(wq_ref[...], e, (((0,), (1,)), ((), ())),
                          preferred_element_type=jnp.float32)
    eqt_ref[0] = _leaky(eqt + bqc_ref[...])


def _project(e, WQ, bQc, WK, bK2, WV, bV2):
    return pl.pallas_call(
        _project_body,
        grid=(NW,),
        in_specs=[
            pl.BlockSpec((NPW, D), lambda g: (g, 0)),
            pl.BlockSpec((D, H), lambda g: (0, 0)),
            pl.BlockSpec((H, 1), lambda g: (0, 0)),
            pl.BlockSpec((D, H), lambda g: (0, 0)),
            pl.BlockSpec((1, H), lambda g: (0, 0)),
            pl.BlockSpec((D, H), lambda g: (0, 0)),
            pl.BlockSpec((1, H), lambda g: (0, 0)),
        ],
        out_specs=[
            pl.BlockSpec((1, H, NPW), lambda g: (g, 0, 0)),
            pl.BlockSpec((NPW, H), lambda g: (g, 0)),
            pl.BlockSpec((NPW, H), lambda g: (g, 0)),
        ],
        out_shape=[
            jax.ShapeDtypeStruct((NW, H, NPW), jnp.float32),
            jax.ShapeDtypeStruct((N, H), jnp.float32),
            jax.ShapeDtypeStruct((N, H), jnp.float32),
        ],
    )(e, WQ, bQc, WK, bK2, WV, bV2)


def _sc_attend_body(ns_hbm, eqt_hbm, ek_hbm, ev_hbm, outt_hbm,
                    idx_v, rows_v, eqt_v, pooledt_v, sem):
    wid = lax.axis_index("s") * 2 + lax.axis_index("c")
    pltpu.sync_copy(ns_hbm.at[wid], idx_v)
    pltpu.sync_copy(eqt_hbm.at[wid], eqt_v)

    lanes = lax.iota(jnp.int32, 16)
    row_base = lanes * K

    for g in range(NGROUPS):
        g16 = g * GROUP

        def gather_rows(table_hbm):
            copies = [
                pltpu.async_copy(
                    table_hbm.at[idx_v.at[4 * g + j]],
                    rows_v.at[pl.ds(j * CHUNK, CHUNK)],
                    sem)
                for j in range(4)
            ]
            for c in copies:
                c.wait()

        gather_rows(ev_hbm)

        def score_h(h, s):
            col = jnp.full((16,), 0, jnp.int32) + h
            eqt = eqt_v[h, pl.ds(g16, GROUP)]
            return tuple(
                s[k] + eqt * plsc.load_gather(rows_v, [row_base + k, col])
                for k in range(K)
            )

        s0 = tuple(jnp.zeros((16,), jnp.float32) for _ in range(K))
        s = lax.fori_loop(0, H, score_h, s0)

        m = s[0]
        for k in range(1, K):
            m = jnp.maximum(m, s[k])
        e = [jnp.exp(s[k] - m) for k in range(K)]
        den = e[0]
        for k in range(1, K):
            den = den + e[k]
        inv = 1.0 / den
        a = [e[k] * inv for k in range(K)]

        gather_rows(ek_hbm)

        def pool_h(h, carry):
            col = jnp.full((16,), 0, jnp.int32) + h
            acc = a[0] * plsc.load_gather(rows_v, [row_base, col])
            for k in range(1, K):
                acc = acc + a[k] * plsc.load_gather(rows_v, [row_base + k, col])
            pooledt_v[h, pl.ds(g16, GROUP)] = acc
            return carry

        lax.fori_loop(0, H, pool_h, 0)

    pltpu.sync_copy(pooledt_v, outt_hbm.at[wid])


def _sc_attend(ns_r, eqt_blocks, ek, ev):
    mesh = plsc.VectorSubcoreMesh(core_axis_name="c", subcore_axis_name="s")
    run = functools.partial(
        pl.kernel,
        mesh=mesh,
        compiler_params=pltpu.CompilerParams(needs_layout_passes=False),
        out_type=jax.ShapeDtypeStruct((NW, H, NPW), jnp.float32),
        scratch_types=[
            pltpu.VMEM((NCHUNK, CHUNK), jnp.int32),
            pltpu.VMEM((ROWS, H), jnp.float32),
            pltpu.VMEM((H, NPW), jnp.float32),
            pltpu.VMEM((H, NPW), jnp.float32),
            pltpu.SemaphoreType.DMA,
        ],
    )(_sc_attend_body)
    return run(ns_r, eqt_blocks, ek, ev)


def _post_body(e_ref, pt_ref, w1a_ref, w1b_ref, b1_ref,
               gamma_ref, beta_ref, mm_ref, mv_ref, out_ref):
    e = e_ref[...]
    hidden = _leaky(
        jnp.dot(e, w1a_ref[...], preferred_element_type=jnp.float32)
        + lax.dot_general(pt_ref[0], w1b_ref[...], (((0,), (0,)), ((), ())),
                          preferred_element_type=jnp.float32)
        + b1_ref[...])
    nrm = jnp.sqrt(jnp.sum(hidden * hidden, axis=1, keepdims=True))
    normalized = hidden / (nrm + 1e-6)
    out_ref[...] = (gamma_ref[...] * (normalized - mm_ref[...])
                    / jnp.sqrt(mv_ref[...] + 1e-3) + beta_ref[...])


def _post(e, pooledt, W1a, W1b, b12, gamma2, beta2, mm2, mv2):
    return pl.pallas_call(
        _post_body,
        grid=(NW,),
        in_specs=[
            pl.BlockSpec((NPW, D), lambda g: (g, 0)),
            pl.BlockSpec((1, H, NPW), lambda g: (g, 0, 0)),
            pl.BlockSpec((D, H), lambda g: (0, 0)),
            pl.BlockSpec((H, H), lambda g: (0, 0)),
            pl.BlockSpec((1, H), lambda g: (0, 0)),
            pl.BlockSpec((1, H), lambda g: (0, 0)),
            pl.BlockSpec((1, H), lambda g: (0, 0)),
            pl.BlockSpec((1, H), lambda g: (0, 0)),
            pl.BlockSpec((1, H), lambda g: (0, 0)),
        ],
        out_specs=pl.BlockSpec((NPW, H), lambda g: (g, 0)),
        out_shape=jax.ShapeDtypeStruct((N, H), jnp.float32),
    )(e, pooledt, W1a, W1b, b12, gamma2, beta2, mm2, mv2)


def kernel(embeddings, weights, neighbor_set, WQ, bQ, WK, bK, WV, bV, W1, b1,
           gamma, beta, moving_mean, moving_var):
    e = embeddings[0]
    ns_r = neighbor_set[0].reshape(NW, NCHUNK, CHUNK)

    eqt_blocks, ek, ev = _project(
        e, WQ, bQ.reshape(H, 1), WK, bK.reshape(1, H), WV, bV.reshape(1, H))

    pooledt = _sc_attend(ns_r, eqt_blocks, ek, ev)

    out = _post(
        e, pooledt, W1[:D], W1[D:], b1.reshape(1, H),
        gamma.reshape(1, H), beta.reshape(1, H),
        moving_mean.reshape(1, H), moving_var.reshape(1, H))
    return out.reshape(1, N, H)


# speedup vs baseline: 9.5172x; 4.6695x over previous
import functools

import jax
import jax.numpy as jnp
from jax import lax
from jax.experimental import pallas as pl
from jax.experimental.pallas import tpu as pltpu
from jax.experimental.pallas import tpu_sc as plsc

N = 4096
K = 32
D = 128
H = 128
NW = 32
NPW = N // NW
GROUP = 16
NGROUPS = NPW // GROUP
ROWS = GROUP * K
CHUNK = 128
NCHUNK = NPW * K // CHUNK


def _leaky(x):
    return jnp.where(x >= 0, x, 0.3 * x)


def _project_body(e_ref, wq_ref, bqc_ref, wk_ref, bk_ref, wv_ref, bv_ref,
                  eqt_ref, ek_ref, ev_ref):
    e = e_ref[...]
    ek_ref[...] = _leaky(
        jnp.dot(e, wk_ref[...], preferred_element_type=jnp.float32) + bk_ref[...])
    ev_ref[...] = _leaky(
        jnp.dot(e, wv_ref[...], preferred_element_type=jnp.float32) + bv_ref[...])
    eqt = lax.dot_general(wq_ref[...], e, (((0,), (1,)), ((), ())),
                          preferred_element_type=jnp.float32)
    eqt_ref[0] = _leaky(eqt + bqc_ref[...])


def _project(e, WQ, bQc, WK, bK2, WV, bV2):
    return pl.pallas_call(
        _project_body,
        grid=(NW,),
        in_specs=[
            pl.BlockSpec((NPW, D), lambda g: (g, 0)),
            pl.BlockSpec((D, H), lambda g: (0, 0)),
            pl.BlockSpec((H, 1), lambda g: (0, 0)),
            pl.BlockSpec((D, H), lambda g: (0, 0)),
            pl.BlockSpec((1, H), lambda g: (0, 0)),
            pl.BlockSpec((D, H), lambda g: (0, 0)),
            pl.BlockSpec((1, H), lambda g: (0, 0)),
        ],
        out_specs=[
            pl.BlockSpec((1, H, NPW), lambda g: (g, 0, 0)),
            pl.BlockSpec((NPW, H), lambda g: (g, 0)),
            pl.BlockSpec((NPW, H), lambda g: (g, 0)),
        ],
        out_shape=[
            jax.ShapeDtypeStruct((NW, H, NPW), jnp.float32),
            jax.ShapeDtypeStruct((N, H), jnp.float32),
            jax.ShapeDtypeStruct((N, H), jnp.float32),
        ],
    )(e, WQ, bQc, WK, bK2, WV, bV2)


NOCT = 4
KO = K // NOCT


def _sc_attend_body(ns_hbm, eqt_hbm, ek_hbm, ev_hbm, outt_hbm,
                    idx_v, rows_v, eqt_v, eqtrot_v, pooledtrot_v,
                    sem_ev, sem_ek):
    wid = lax.axis_index("s") * 2 + lax.axis_index("c")
    pltpu.sync_copy(ns_hbm.at[wid], idx_v)
    pltpu.sync_copy(eqt_hbm.at[wid], eqt_v)

    lanes = lax.iota(jnp.int32, 16)

    def fire(table_hbm, g, c, sem):
        return pltpu.async_copy(
            table_hbm.at[idx_v.at[NOCT * g + c]],
            rows_v.at[pl.ds(c * CHUNK, CHUNK)],
            sem)

    def rot_h(h, carry):
        hrot = (h + lanes) & (H - 1)
        for s8 in range(NPW // 16):
            col = jnp.full((16,), s8 * 16, jnp.int32) + lanes
            eqtrot_v[h, pl.ds(s8 * 16, 16)] = plsc.load_gather(
                eqt_v, [hrot, col])
        return carry

    lax.fori_loop(0, H, rot_h, 0)

    ev_pending = [fire(ev_hbm, 0, c, sem_ev) for c in range(NOCT)]

    for g in range(NGROUPS):
        g16 = g * GROUP

        s = []
        ek_pending = []
        for c in range(NOCT):
            ev_pending[c].wait()

            def score_h(h, sc, c=c):
                colrot = (jnp.full((16,), 0, jnp.int32) + h + lanes) & (H - 1)
                eqt = eqtrot_v[h, pl.ds(g16, GROUP)]
                return tuple(
                    sc[kk] + eqt * plsc.load_gather(
                        rows_v, [lanes * KO + (c * CHUNK + kk), colrot])
                    for kk in range(KO)
                )

            s0 = tuple(jnp.zeros((16,), jnp.float32) for _ in range(KO))
            s.extend(lax.fori_loop(0, H, score_h, s0))
            ek_pending.append(fire(ek_hbm, g, c, sem_ek))

        m = s[0]
        for k in range(1, K):
            m = jnp.maximum(m, s[k])
        e = [jnp.exp(s[k] - m) for k in range(K)]
        den = e[0]
        for k in range(1, K):
            den = den + e[k]
        inv = 1.0 / den
        a = [e[k] * inv for k in range(K)]

        for c in range(NOCT):
            ek_pending[c].wait()
            ac = a[c * KO:(c + 1) * KO]

            def pool_h(h, carry, c=c, ac=ac):
                colrot = (jnp.full((16,), 0, jnp.int32) + h + lanes) & (H - 1)
                acc = ac[0] * plsc.load_gather(
                    rows_v, [lanes * KO + c * CHUNK, colrot])
                for kk in range(1, KO):
                    acc = acc + ac[kk] * plsc.load_gather(
                        rows_v, [lanes * KO + (c * CHUNK + kk), colrot])
                if c == 0:
                    pooledtrot_v[h, pl.ds(g16, GROUP)] = acc
                else:
                    pooledtrot_v[h, pl.ds(g16, GROUP)] += acc
                return carry

            lax.fori_loop(0, H, pool_h, 0)
            if g + 1 < NGROUPS:
                ev_pending[c] = fire(ev_hbm, g + 1, c, sem_ev)

    def unrot_h(h, carry):
        hrot = (h - lanes) & (H - 1)
        for s8 in range(NPW // 16):
            col = jnp.full((16,), s8 * 16, jnp.int32) + lanes
            eqt_v[h, pl.ds(s8 * 16, 16)] = plsc.load_gather(
                pooledtrot_v, [hrot, col])
        return carry

    lax.fori_loop(0, H, unrot_h, 0)

    pltpu.sync_copy(eqt_v, outt_hbm.at[wid])


def _sc_attend(ns_r, eqt_blocks, ek, ev):
    mesh = plsc.VectorSubcoreMesh(core_axis_name="c", subcore_axis_name="s")
    run = functools.partial(
        pl.kernel,
        mesh=mesh,
        compiler_params=pltpu.CompilerParams(needs_layout_passes=False),
        out_type=jax.ShapeDtypeStruct((NW, H, NPW), jnp.float32),
        scratch_types=[
            pltpu.VMEM((NCHUNK, CHUNK), jnp.int32),
            pltpu.VMEM((ROWS, H), jnp.float32),
            pltpu.VMEM((H, NPW), jnp.float32),
            pltpu.VMEM((H, NPW), jnp.float32),
            pltpu.VMEM((H, NPW), jnp.float32),
            pltpu.SemaphoreType.DMA,
            pltpu.SemaphoreType.DMA,
        ],
    )(_sc_attend_body)
    return run(ns_r, eqt_blocks, ek, ev)


def _post_body(e_ref, pt_ref, w1a_ref, w1b_ref, b1_ref,
               gamma_ref, beta_ref, mm_ref, mv_ref, out_ref):
    e = e_ref[...]
    hidden = _leaky(
        jnp.dot(e, w1a_ref[...], preferred_element_type=jnp.float32)
        + lax.dot_general(pt_ref[0], w1b_ref[...], (((0,), (0,)), ((), ())),
                          preferred_element_type=jnp.float32)
        + b1_ref[...])
    nrm = jnp.sqrt(jnp.sum(hidden * hidden, axis=1, keepdims=True))
    normalized = hidden / (nrm + 1e-6)
    out_ref[...] = (gamma_ref[...] * (normalized - mm_ref[...])
                    / jnp.sqrt(mv_ref[...] + 1e-3) + beta_ref[...])


def _post(e, pooledt, W1a, W1b, b12, gamma2, beta2, mm2, mv2):
    return pl.pallas_call(
        _post_body,
        grid=(NW,),
        in_specs=[
            pl.BlockSpec((NPW, D), lambda g: (g, 0)),
            pl.BlockSpec((1, H, NPW), lambda g: (g, 0, 0)),
            pl.BlockSpec((D, H), lambda g: (0, 0)),
            pl.BlockSpec((H, H), lambda g: (0, 0)),
            pl.BlockSpec((1, H), lambda g: (0, 0)),
            pl.BlockSpec((1, H), lambda g: (0, 0)),
            pl.BlockSpec((1, H), lambda g: (0, 0)),
            pl.BlockSpec((1, H), lambda g: (0, 0)),
            pl.BlockSpec((1, H), lambda g: (0, 0)),
        ],
        out_specs=pl.BlockSpec((NPW, H), lambda g: (g, 0)),
        out_shape=jax.ShapeDtypeStruct((N, H), jnp.float32),
    )(e, pooledt, W1a, W1b, b12, gamma2, beta2, mm2, mv2)


def kernel(embeddings, weights, neighbor_set, WQ, bQ, WK, bK, WV, bV, W1, b1,
           gamma, beta, moving_mean, moving_var):
    e = embeddings[0]
    ns_r = (neighbor_set[0]
            .reshape(NW, NGROUPS, GROUP, NOCT, KO)
            .transpose(0, 1, 3, 2, 4)
            .reshape(NW, NCHUNK, CHUNK))

    eqt_blocks, ek, ev = _project(
        e, WQ, bQ.reshape(H, 1), WK, bK.reshape(1, H), WV, bV.reshape(1, H))

    pooledt = _sc_attend(ns_r, eqt_blocks, ek, ev)

    out = _post(
        e, pooledt, W1[:D], W1[D:], b1.reshape(1, H),
        gamma.reshape(1, H), beta.reshape(1, H),
        moving_mean.reshape(1, H), moving_var.reshape(1, H))
    return out.reshape(1, N, H)
